# initial kernel scaffold (unmeasured)
import jax
import jax.numpy as jnp
from jax import lax
from jax.experimental import pallas as pl
from jax.experimental.pallas import tpu as pltpu

N_DEV = 8
HQ = 8
DH = 128
SQ = 1024
SKV = 1024
HD = HQ * DH
WINDOW = 128
SCALE = 0.08838834764831843
NEG = -1e9

_sem_signal = getattr(pl, "semaphore_signal", None) or pltpu.semaphore_signal
_sem_wait = getattr(pl, "semaphore_wait", None) or pltpu.semaphore_wait
_CompilerParams = getattr(pltpu, "CompilerParams", None) or pltpu.TPUCompilerParams


def kernel(x, Wq, K_ext, V_ext, Wo):
    def body(x_ref, wq_ref, k_ref, v_ref, wo_ref, out_ref,
             comm, kblk, vblk, qs, ctxs,
             send_sems, recv_sems, kv_sems, credit_sem):
        my = lax.axis_index("i")
        left = lax.rem(my - 1 + N_DEV, N_DEV)
        right = lax.rem(my + 1, N_DEV)

        comm[0, 0] = wq_ref[...]
        comm[0, 1] = wo_ref[...]

        barrier = pltpu.get_barrier_semaphore()
        for nbr in (left, right):
            _sem_signal(barrier, inc=1, device_id=(nbr,),
                        device_id_type=pl.DeviceIdType.MESH)
        _sem_wait(barrier, 2)

        qi = lax.broadcasted_iota(jnp.int32, (SQ, SKV), 0)
        ki = lax.broadcasted_iota(jnp.int32, (SQ, SKV), 1)
        mask = jnp.abs(qi - ki) <= WINDOW

        acc = None
        for h in range(N_DEV):
            slot = h % 2
            j = lax.rem(my - h + N_DEV, N_DEV)

            kcopy = pltpu.make_async_copy(
                k_ref.at[my, :, pl.ds(j * HQ, HQ), :], kblk, kv_sems.at[0])
            vcopy = pltpu.make_async_copy(
                v_ref.at[my, :, pl.ds(j * HQ, HQ), :], vblk, kv_sems.at[1])
            kcopy.start()
            vcopy.start()

            if h > 0:
                recv = pltpu.make_async_remote_copy(
                    src_ref=comm.at[1 - slot], dst_ref=comm.at[slot],
                    send_sem=send_sems.at[h - 1], recv_sem=recv_sems.at[h - 1],
                    device_id=(left,), device_id_type=pl.DeviceIdType.MESH)
                recv.wait_recv()

            rdma = None
            if h < N_DEV - 1:
                if h >= 1:
                    _sem_wait(credit_sem, 1)
                rdma = pltpu.make_async_remote_copy(
                    src_ref=comm.at[slot], dst_ref=comm.at[1 - slot],
                    send_sem=send_sems.at[h], recv_sem=recv_sems.at[h],
                    device_id=(right,), device_id_type=pl.DeviceIdType.MESH)
                rdma.start()

            kcopy.wait()
            vcopy.wait()

            qs[...] = jnp.dot(x_ref[0], comm[slot, 0],
                              preferred_element_type=jnp.float32)

            def head_body(k, _):
                qk = qs[:, pl.ds(k * DH, DH)]
                kk = kblk[:, pl.ds(k, 1), :].reshape(SKV, DH)
                s = lax.dot_general(
                    qk, kk, (((1,), (1,)), ((), ())),
                    preferred_element_type=jnp.float32) * SCALE
                s = jnp.where(mask, s, NEG)
                m = jnp.max(s, axis=1, keepdims=True)
                w = jnp.exp(s - m)
                w = w / jnp.sum(w, axis=1, keepdims=True)
                vk = vblk[:, pl.ds(k, 1), :].reshape(SKV, DH)
                ctxs[:, pl.ds(k * DH, DH)] = jnp.dot(
                    w, vk, preferred_element_type=jnp.float32)
                return 0

            lax.fori_loop(0, HQ, head_body, 0)

            part = jnp.dot(ctxs[...], comm[slot, 1],
                           preferred_element_type=jnp.float32)
            acc = part if acc is None else acc + part

            if rdma is not None:
                rdma.wait_send()
            if h <= N_DEV - 3:
                _sem_signal(credit_sem, inc=1, device_id=(left,),
                            device_id_type=pl.DeviceIdType.MESH)

        out_ref[0] = acc

    return pl.pallas_call(
        body,
        out_shape=jax.ShapeDtypeStruct((1, SQ, HD), jnp.float32),
        in_specs=[
            pl.BlockSpec(memory_space=pltpu.VMEM),
            pl.BlockSpec(memory_space=pltpu.VMEM),
            pl.BlockSpec(memory_space=pltpu.ANY),
            pl.BlockSpec(memory_space=pltpu.ANY),
            pl.BlockSpec(memory_space=pltpu.VMEM),
        ],
        out_specs=pl.BlockSpec(memory_space=pltpu.VMEM),
        scratch_shapes=[
            pltpu.VMEM((2, 2, SQ, HD), jnp.float32),
            pltpu.VMEM((SKV, HQ, DH), jnp.float32),
            pltpu.VMEM((SKV, HQ, DH), jnp.float32),
            pltpu.VMEM((SQ, HD), jnp.float32),
            pltpu.VMEM((SQ, HD), jnp.float32),
            pltpu.SemaphoreType.DMA((N_DEV,)),
            pltpu.SemaphoreType.DMA((N_DEV,)),
            pltpu.SemaphoreType.DMA((2,)),
            pltpu.SemaphoreType.REGULAR,
        ],
        compiler_params=_CompilerParams(collective_id=0),
    )(x, Wq, K_ext, V_ext, Wo)


# baseline (device time: 690744 ns/iter reference)
import jax
import jax.numpy as jnp
from jax import lax
from jax.experimental import pallas as pl
from jax.experimental.pallas import tpu as pltpu

import os

N_DEV = 8
N_HOPS = int(os.environ.get("K_N_HOPS", str(N_DEV)))
DO_RDMA = os.environ.get("K_DO_RDMA", "1") == "1"
DO_BARRIER = os.environ.get("K_DO_BARRIER", "1") == "1" and DO_RDMA
DO_SEND = os.environ.get("K_DO_SEND", "1") == "1" and DO_RDMA
DO_CREDIT = os.environ.get("K_DO_CREDIT", "1") == "1" and DO_RDMA
INTERPRET = os.environ.get("K_INTERPRET", "0") == "1"
HQ = 8
DH = 128
SQ = 1024
SKV = 1024
HD = HQ * DH
WINDOW = 128
SCALE = 0.08838834764831843
NEG = -1e9

_sem_signal = getattr(pl, "semaphore_signal", None) or pltpu.semaphore_signal
_sem_wait = getattr(pl, "semaphore_wait", None) or pltpu.semaphore_wait
_CompilerParams = getattr(pltpu, "CompilerParams", None) or pltpu.TPUCompilerParams


def kernel(x, Wq, K_ext, V_ext, Wo):
    def body(x_ref, wq_ref, k_ref, v_ref, wo_ref, out_ref,
             comm, priv, kblk, vblk, qs,
             send_sems, recv_sems, kv_sems, credit_sem):
        my = lax.axis_index("i")
        left = lax.rem(my - 1 + N_DEV, N_DEV)
        right = lax.rem(my + 1, N_DEV)

        comm[0, 0] = wq_ref[...]
        comm[0, 1] = wo_ref[...]

        if DO_BARRIER:
            barrier = pltpu.get_barrier_semaphore()
            for nbr in (left, right):
                _sem_signal(barrier, inc=1, device_id=(nbr,),
                            device_id_type=pl.DeviceIdType.MESH)
            _sem_wait(barrier, 2)

        out_ref[0] = jnp.zeros((SQ, HD), jnp.float32)

        for h in range(N_HOPS):
            slot = h % 2
            j = lax.rem(my - h + N_DEV, N_DEV)

            kcopy = pltpu.make_async_copy(
                k_ref.at[my, :, pl.ds(j * HQ, HQ), :], kblk, kv_sems.at[0])
            vcopy = pltpu.make_async_copy(
                v_ref.at[my, :, pl.ds(j * HQ, HQ), :], vblk, kv_sems.at[1])
            kcopy.start()
            vcopy.start()

            if h > 0 and DO_SEND:
                recv = pltpu.make_async_remote_copy(
                    src_ref=comm.at[1 - slot], dst_ref=comm.at[slot],
                    send_sem=send_sems.at[h - 1], recv_sem=recv_sems.at[h - 1],
                    device_id=(left,), device_id_type=pl.DeviceIdType.MESH)
                recv.wait_recv()

            rdma = None
            if DO_SEND and h < N_DEV - 1:
                if h >= 1 and DO_CREDIT:
                    _sem_wait(credit_sem, 1)
                rdma = pltpu.make_async_remote_copy(
                    src_ref=comm.at[slot], dst_ref=comm.at[1 - slot],
                    send_sem=send_sems.at[h], recv_sem=recv_sems.at[h],
                    device_id=(right,), device_id_type=pl.DeviceIdType.MESH)
                rdma.start()

            kcopy.wait()
            vcopy.wait()

            priv[...] = comm[slot, 0]

            qs[...] = jnp.dot(x_ref[0], priv[...],
                              preferred_element_type=jnp.float32)

            priv[...] = comm[slot, 1]

            def head_body(k, _):
                qk = qs[:, pl.ds(k * DH, DH)]
                kk = kblk[:, pl.ds(k, 1), :].reshape(SKV, DH)
                s = lax.dot_general(
                    qk, kk, (((1,), (1,)), ((), ())),
                    preferred_element_type=jnp.float32) * SCALE
                qi = lax.broadcasted_iota(jnp.int32, (SQ, SKV), 0)
                ki = lax.broadcasted_iota(jnp.int32, (SQ, SKV), 1)
                s = jnp.where(jnp.abs(qi - ki) <= WINDOW, s, NEG)
                m = jnp.max(s, axis=1, keepdims=True)
                w = jnp.exp(s - m)
                w = w / jnp.sum(w, axis=1, keepdims=True)
                vk = vblk[:, pl.ds(k, 1), :].reshape(SKV, DH)
                ctx = jnp.dot(w, vk, preferred_element_type=jnp.float32)
                out_ref[0] += jnp.dot(
                    ctx, priv[pl.ds(k * DH, DH), :],
                    preferred_element_type=jnp.float32)
                return 0

            lax.fori_loop(0, HQ, head_body, 0)

            if rdma is not None:
                rdma.wait_send()
            if DO_CREDIT and h <= N_DEV - 3:
                _sem_signal(credit_sem, inc=1, device_id=(left,),
                            device_id_type=pl.DeviceIdType.MESH)

    return pl.pallas_call(
        body,
        out_shape=jax.ShapeDtypeStruct((1, SQ, HD), jnp.float32),
        in_specs=[
            pl.BlockSpec(memory_space=pltpu.VMEM),
            pl.BlockSpec(memory_space=pltpu.VMEM),
            pl.BlockSpec(memory_space=pl.ANY),
            pl.BlockSpec(memory_space=pl.ANY),
            pl.BlockSpec(memory_space=pltpu.VMEM),
        ],
        out_specs=pl.BlockSpec(memory_space=pltpu.VMEM),
        scratch_shapes=[
            pltpu.VMEM((2, 2, SQ, HD), jnp.float32),
            pltpu.VMEM((SQ, HD), jnp.float32),
            pltpu.VMEM((SKV, HQ, DH), jnp.float32),
            pltpu.VMEM((SKV, HQ, DH), jnp.float32),
            pltpu.VMEM((SQ, HD), jnp.float32),
            pltpu.SemaphoreType.DMA((N_DEV,)),
            pltpu.SemaphoreType.DMA((N_DEV,)),
            pltpu.SemaphoreType.DMA((2,)),
            pltpu.SemaphoreType.REGULAR,
        ],
        compiler_params=_CompilerParams(
            collective_id=0 if DO_BARRIER else None,
            vmem_limit_bytes=110 * 1024 * 1024),
        interpret=(pltpu.InterpretParams(detect_races=True)
                   if INTERPRET else False),
    )(x, Wq, K_ext, V_ext, Wo)


# device time: 373594 ns/iter; 1.8489x vs baseline; 1.8489x over previous
import jax
import jax.numpy as jnp
from jax import lax
from jax.experimental import pallas as pl
from jax.experimental.pallas import tpu as pltpu

import os

N_DEV = 8
N_HOPS = int(os.environ.get("K_N_HOPS", str(N_DEV)))
DO_RDMA = os.environ.get("K_DO_RDMA", "1") == "1"
DO_BARRIER = os.environ.get("K_DO_BARRIER", "1") == "1" and DO_RDMA
DO_SEND = os.environ.get("K_DO_SEND", "1") == "1" and DO_RDMA
DO_CREDIT = os.environ.get("K_DO_CREDIT", "1") == "1" and DO_RDMA
INTERPRET = os.environ.get("K_INTERPRET", "0") == "1"
HQ = 8
DH = 128
SQ = 1024
SKV = 1024
HD = HQ * DH
WINDOW = 128
SCALE = 0.08838834764831843
NEG = -1e9

_sem_signal = getattr(pl, "semaphore_signal", None) or pltpu.semaphore_signal
_sem_wait = getattr(pl, "semaphore_wait", None) or pltpu.semaphore_wait
_CompilerParams = getattr(pltpu, "CompilerParams", None) or pltpu.TPUCompilerParams


def kernel(x, Wq, K_ext, V_ext, Wo):
    def body(x_ref, wq_ref, k_ref, v_ref, wo_ref, out_ref,
             comm, priv, xb, kblk, vblk, qs, ctxs,
             send_sems, recv_sems, kv_sems, credit_sem):
        my = lax.axis_index("i")
        left = lax.rem(my - 1 + N_DEV, N_DEV)
        right = lax.rem(my + 1, N_DEV)

        comm[0, 0] = wq_ref[...].astype(jnp.bfloat16)
        comm[0, 1] = wo_ref[...].astype(jnp.bfloat16)
        xb[...] = x_ref[0].astype(jnp.bfloat16)

        if DO_BARRIER:
            barrier = pltpu.get_barrier_semaphore()
            for nbr in (left, right):
                _sem_signal(barrier, inc=1, device_id=(nbr,),
                            device_id_type=pl.DeviceIdType.MESH)
            _sem_wait(barrier, 2)

        out_ref[0] = jnp.zeros((SQ, HD), jnp.float32)

        for h in range(N_HOPS):
            slot = h % 2
            j = lax.rem(my - h + N_DEV, N_DEV)

            kcopy = pltpu.make_async_copy(
                k_ref.at[my, :, pl.ds(j * HQ, HQ), :], kblk, kv_sems.at[0])
            vcopy = pltpu.make_async_copy(
                v_ref.at[my, :, pl.ds(j * HQ, HQ), :], vblk, kv_sems.at[1])
            kcopy.start()
            vcopy.start()

            if h > 0 and DO_SEND:
                recv = pltpu.make_async_remote_copy(
                    src_ref=comm.at[1 - slot], dst_ref=comm.at[slot],
                    send_sem=send_sems.at[h - 1], recv_sem=recv_sems.at[h - 1],
                    device_id=(left,), device_id_type=pl.DeviceIdType.MESH)
                recv.wait_recv()

            rdma = None
            if DO_SEND and h < N_DEV - 1:
                if h >= 1 and DO_CREDIT:
                    _sem_wait(credit_sem, 1)
                rdma = pltpu.make_async_remote_copy(
                    src_ref=comm.at[slot], dst_ref=comm.at[1 - slot],
                    send_sem=send_sems.at[h], recv_sem=recv_sems.at[h],
                    device_id=(right,), device_id_type=pl.DeviceIdType.MESH)
                rdma.start()

            kcopy.wait()
            vcopy.wait()

            priv[...] = comm[slot, 0]

            qs[...] = jnp.dot(xb[...], priv[...],
                              preferred_element_type=jnp.float32
                              ).astype(jnp.bfloat16)

            priv[...] = comm[slot, 1]

            def head_body(k, _):
                qk = qs[:, pl.ds(k * DH, DH)]
                kk = kblk[:, pl.ds(k, 1), :].reshape(SKV, DH)
                s = lax.dot_general(
                    qk, kk.astype(jnp.bfloat16), (((1,), (1,)), ((), ())),
                    preferred_element_type=jnp.float32) * SCALE
                qi = lax.broadcasted_iota(jnp.int32, (SQ, SKV), 0)
                ki = lax.broadcasted_iota(jnp.int32, (SQ, SKV), 1)
                s = jnp.where(jnp.abs(qi - ki) <= WINDOW, s, NEG)
                m = jnp.max(s, axis=1, keepdims=True)
                w = jnp.exp(s - m)
                w = (w / jnp.sum(w, axis=1, keepdims=True)).astype(jnp.bfloat16)
                vk = vblk[:, pl.ds(k, 1), :].reshape(SKV, DH)
                ctxs[:, pl.ds(k * DH, DH)] = jnp.dot(
                    w, vk.astype(jnp.bfloat16),
                    preferred_element_type=jnp.float32).astype(jnp.bfloat16)
                return 0

            lax.fori_loop(0, HQ, head_body, 0)

            out_ref[0] += jnp.dot(ctxs[...], priv[...],
                                  preferred_element_type=jnp.float32)

            if rdma is not None:
                rdma.wait_send()
            if DO_CREDIT and h <= N_DEV - 3:
                _sem_signal(credit_sem, inc=1, device_id=(left,),
                            device_id_type=pl.DeviceIdType.MESH)

    return pl.pallas_call(
        body,
        out_shape=jax.ShapeDtypeStruct((1, SQ, HD), jnp.float32),
        in_specs=[
            pl.BlockSpec(memory_space=pltpu.VMEM),
            pl.BlockSpec(memory_space=pltpu.VMEM),
            pl.BlockSpec(memory_space=pl.ANY),
            pl.BlockSpec(memory_space=pl.ANY),
            pl.BlockSpec(memory_space=pltpu.VMEM),
        ],
        out_specs=pl.BlockSpec(memory_space=pltpu.VMEM),
        scratch_shapes=[
            pltpu.VMEM((2, 2, SQ, HD), jnp.bfloat16),
            pltpu.VMEM((SQ, HD), jnp.bfloat16),
            pltpu.VMEM((SQ, HD), jnp.bfloat16),
            pltpu.VMEM((SKV, HQ, DH), jnp.float32),
            pltpu.VMEM((SKV, HQ, DH), jnp.float32),
            pltpu.VMEM((SQ, HD), jnp.bfloat16),
            pltpu.VMEM((SQ, HD), jnp.bfloat16),
            pltpu.SemaphoreType.DMA((N_DEV,)),
            pltpu.SemaphoreType.DMA((N_DEV,)),
            pltpu.SemaphoreType.DMA((2,)),
            pltpu.SemaphoreType.REGULAR,
        ],
        compiler_params=_CompilerParams(
            collective_id=0 if DO_BARRIER else None,
            vmem_limit_bytes=110 * 1024 * 1024),
        interpret=(pltpu.InterpretParams(detect_races=True)
                   if INTERPRET else False),
    )(x, Wq, K_ext, V_ext, Wo)


# device time: 368184 ns/iter; 1.8761x vs baseline; 1.0147x over previous
import jax
import jax.numpy as jnp
from jax import lax
from jax.experimental import pallas as pl
from jax.experimental.pallas import tpu as pltpu

import os

N_DEV = 8
N_HOPS = int(os.environ.get("K_N_HOPS", str(N_DEV)))
DO_RDMA = os.environ.get("K_DO_RDMA", "1") == "1"
DO_BARRIER = os.environ.get("K_DO_BARRIER", "1") == "1" and DO_RDMA
DO_SEND = os.environ.get("K_DO_SEND", "1") == "1" and DO_RDMA
DO_CREDIT = os.environ.get("K_DO_CREDIT", "1") == "1" and DO_RDMA
INTERPRET = os.environ.get("K_INTERPRET", "0") == "1"
HQ = 8
DH = 128
SQ = 1024
SKV = 1024
HD = HQ * DH
WINDOW = 128
SCALE = 0.08838834764831843
NEG = -1e9

_sem_signal = getattr(pl, "semaphore_signal", None) or pltpu.semaphore_signal
_sem_wait = getattr(pl, "semaphore_wait", None) or pltpu.semaphore_wait
_CompilerParams = getattr(pltpu, "CompilerParams", None) or pltpu.TPUCompilerParams


def kernel(x, Wq, K_ext, V_ext, Wo):
    def body(x_ref, wq_ref, k_ref, v_ref, wo_ref, out_ref,
             comm, priv, xb, kblk, vblk, qs, ctxs, mbias,
             send_sems, recv_sems, kv_sems, credit_sem):
        my = lax.axis_index("i")
        left = lax.rem(my - 1 + N_DEV, N_DEV)
        right = lax.rem(my + 1, N_DEV)

        comm[0, 0] = wq_ref[...].astype(jnp.bfloat16)
        comm[0, 1] = wo_ref[...].astype(jnp.bfloat16)
        xb[...] = x_ref[0].astype(jnp.bfloat16)
        qi = lax.broadcasted_iota(jnp.int32, (SQ, SKV), 0)
        ki = lax.broadcasted_iota(jnp.int32, (SQ, SKV), 1)
        mbias[...] = jnp.where(jnp.abs(qi - ki) <= WINDOW, 0.0, NEG)

        if DO_BARRIER:
            barrier = pltpu.get_barrier_semaphore()
            for nbr in (left, right):
                _sem_signal(barrier, inc=1, device_id=(nbr,),
                            device_id_type=pl.DeviceIdType.MESH)
            _sem_wait(barrier, 2)

        out_ref[0] = jnp.zeros((SQ, HD), jnp.float32)

        for h in range(N_HOPS):
            slot = h % 2
            j = lax.rem(my - h + N_DEV, N_DEV)

            kcopy = pltpu.make_async_copy(
                k_ref.at[my, :, pl.ds(j * HQ, HQ), :], kblk, kv_sems.at[0])
            vcopy = pltpu.make_async_copy(
                v_ref.at[my, :, pl.ds(j * HQ, HQ), :], vblk, kv_sems.at[1])
            kcopy.start()
            vcopy.start()

            if h > 0 and DO_SEND:
                recv = pltpu.make_async_remote_copy(
                    src_ref=comm.at[1 - slot], dst_ref=comm.at[slot],
                    send_sem=send_sems.at[h - 1], recv_sem=recv_sems.at[h - 1],
                    device_id=(left,), device_id_type=pl.DeviceIdType.MESH)
                recv.wait_recv()

            rdma = None
            if DO_SEND and h < N_DEV - 1:
                if h >= 1 and DO_CREDIT:
                    _sem_wait(credit_sem, 1)
                rdma = pltpu.make_async_remote_copy(
                    src_ref=comm.at[slot], dst_ref=comm.at[1 - slot],
                    send_sem=send_sems.at[h], recv_sem=recv_sems.at[h],
                    device_id=(right,), device_id_type=pl.DeviceIdType.MESH)
                rdma.start()

            kcopy.wait()
            vcopy.wait()

            priv[...] = comm[slot, 0]

            qs[...] = (jnp.dot(xb[...], priv[...],
                               preferred_element_type=jnp.float32)
                       * SCALE).astype(jnp.bfloat16)

            priv[...] = comm[slot, 1]

            def head_body(k, _):
                qk = qs[:, pl.ds(k * DH, DH)]
                kk = kblk[:, pl.ds(k, 1), :].reshape(SKV, DH)
                s = lax.dot_general(
                    qk, kk.astype(jnp.bfloat16), (((1,), (1,)), ((), ())),
                    preferred_element_type=jnp.float32) + mbias[...]
                w = jnp.exp(s)
                denom = jnp.sum(w, axis=1, keepdims=True)
                vk = vblk[:, pl.ds(k, 1), :].reshape(SKV, DH)
                ctx = jnp.dot(
                    w.astype(jnp.bfloat16), vk.astype(jnp.bfloat16),
                    preferred_element_type=jnp.float32) / denom
                ctxs[:, pl.ds(k * DH, DH)] = ctx.astype(jnp.bfloat16)
                return 0

            lax.fori_loop(0, HQ, head_body, 0)

            out_ref[0] += jnp.dot(ctxs[...], priv[...],
                                  preferred_element_type=jnp.float32)

            if rdma is not None:
                rdma.wait_send()
            if DO_CREDIT and h <= N_DEV - 3:
                _sem_signal(credit_sem, inc=1, device_id=(left,),
                            device_id_type=pl.DeviceIdType.MESH)

    return pl.pallas_call(
        body,
        out_shape=jax.ShapeDtypeStruct((1, SQ, HD), jnp.float32),
        in_specs=[
            pl.BlockSpec(memory_space=pltpu.VMEM),
            pl.BlockSpec(memory_space=pltpu.VMEM),
            pl.BlockSpec(memory_space=pl.ANY),
            pl.BlockSpec(memory_space=pl.ANY),
            pl.BlockSpec(memory_space=pltpu.VMEM),
        ],
        out_specs=pl.BlockSpec(memory_space=pltpu.VMEM),
        scratch_shapes=[
            pltpu.VMEM((2, 2, SQ, HD), jnp.bfloat16),
            pltpu.VMEM((SQ, HD), jnp.bfloat16),
            pltpu.VMEM((SQ, HD), jnp.bfloat16),
            pltpu.VMEM((SKV, HQ, DH), jnp.float32),
            pltpu.VMEM((SKV, HQ, DH), jnp.float32),
            pltpu.VMEM((SQ, HD), jnp.bfloat16),
            pltpu.VMEM((SQ, HD), jnp.bfloat16),
            pltpu.VMEM((SQ, SKV), jnp.float32),
            pltpu.SemaphoreType.DMA((N_DEV,)),
            pltpu.SemaphoreType.DMA((N_DEV,)),
            pltpu.SemaphoreType.DMA((2,)),
            pltpu.SemaphoreType.REGULAR,
        ],
        compiler_params=_CompilerParams(
            collective_id=0 if DO_BARRIER else None,
            vmem_limit_bytes=110 * 1024 * 1024),
        interpret=(pltpu.InterpretParams(detect_races=True)
                   if INTERPRET else False),
    )(x, Wq, K_ext, V_ext, Wo)
